# baseline (speedup 1.0000x reference)
import jax
import jax.numpy as jnp
from jax import lax
from jax.experimental import pallas as pl
from jax.experimental.pallas import tpu as pltpu
from jax.experimental.pallas import tpu_sc as plsc

ROWS = 16384
COLS = 2048
LANES = 16
NUM_WORKERS = 32
ROWS_PER_W = ROWS // NUM_WORKERS
BLK = 16
STEPS = ROWS_PER_W // BLK
CHUNKS = COLS // LANES


def _permute_body(x_hbm, perm_hbm, out_hbm, perm_v, in_v, out_v):
    core = lax.axis_index("c")
    sub = lax.axis_index("s")
    wid = sub * 2 + core
    elem0 = wid * ROWS_PER_W * COLS

    pltpu.sync_copy(perm_hbm, perm_v)

    def step(s, carry):
        base = elem0 + s * BLK * COLS
        pltpu.sync_copy(x_hbm.at[pl.ds(base, BLK * COLS)], in_v)

        def chunk(c, inner):
            col = c * LANES
            idx = perm_v[pl.ds(col, LANES)]
            for r in range(BLK):
                vals = plsc.load_gather(in_v, [idx + (r * COLS)])
                out_v[pl.ds(col + r * COLS, LANES)] = vals
            return inner

        lax.fori_loop(0, CHUNKS, chunk, 0)
        pltpu.sync_copy(out_v, out_hbm.at[pl.ds(base, BLK * COLS)])
        return carry

    lax.fori_loop(0, STEPS, step, 0)


@jax.jit
def _permute(x, perm):
    mesh = plsc.VectorSubcoreMesh(core_axis_name="c", subcore_axis_name="s")
    run = pl.kernel(
        _permute_body,
        mesh=mesh,
        compiler_params=pltpu.CompilerParams(needs_layout_passes=False),
        out_type=jax.ShapeDtypeStruct((ROWS * COLS,), jnp.float32),
        scratch_types=[
            pltpu.VMEM((COLS,), jnp.int32),
            pltpu.VMEM((BLK * COLS,), jnp.float32),
            pltpu.VMEM((BLK * COLS,), jnp.float32),
        ],
    )
    return run(x.reshape(-1), perm).reshape(ROWS, COLS)


def kernel(x, perm, perm_inv):
    out = _permute(x, perm.astype(jnp.int32))
    return (out, 0)


# speedup vs baseline: 1.1421x; 1.1421x over previous
import jax
import jax.numpy as jnp
from jax import lax
from jax.experimental import pallas as pl
from jax.experimental.pallas import tpu as pltpu
from jax.experimental.pallas import tpu_sc as plsc

ROWS = 16384
COLS = 2048
LANES = 16
NUM_WORKERS = 32
ROWS_PER_W = ROWS // NUM_WORKERS
BLK = 8
BLKE = BLK * COLS
STEPS = ROWS_PER_W // BLK
NPAIR = STEPS // 2
CHUNKS = COLS // LANES


def _permute_body(x_hbm, perm_hbm, out_hbm, perm_v,
                  in_a, in_b, out_a, out_b,
                  isem_a, isem_b, osem_a, osem_b):
    core = lax.axis_index("c")
    sub = lax.axis_index("s")
    wid = sub * 2 + core
    elem0 = wid * ROWS_PER_W * COLS

    pltpu.sync_copy(perm_hbm, perm_v)

    bufs = ((in_a, out_a, isem_a, osem_a), (in_b, out_b, isem_b, osem_b))

    for b in range(2):
        pltpu.make_async_copy(
            x_hbm.at[pl.ds(elem0 + b * BLKE, BLKE)], bufs[b][0], bufs[b][2]
        ).start()

    def pair(p, carry):
        for b in range(2):
            in_v, out_v, isem, osem = bufs[b]
            base = elem0 + (p * 2 + b) * BLKE
            pltpu.make_async_copy(
                x_hbm.at[pl.ds(base, BLKE)], in_v, isem).wait()

            @pl.when(p >= 1)
            def _drain():
                pltpu.make_async_copy(
                    out_v, out_hbm.at[pl.ds(base - 2 * BLKE, BLKE)], osem
                ).wait()

            def chunk(c, inner):
                col = c * LANES
                idx = perm_v[pl.ds(col, LANES)]
                for r in range(BLK):
                    vals = plsc.load_gather(in_v, [idx + (r * COLS)])
                    out_v[pl.ds(col + r * COLS, LANES)] = vals
                return inner

            lax.fori_loop(0, CHUNKS, chunk, 0)

            pltpu.make_async_copy(
                out_v, out_hbm.at[pl.ds(base, BLKE)], osem).start()

            @pl.when(p + 1 < NPAIR)
            def _prefetch():
                pltpu.make_async_copy(
                    x_hbm.at[pl.ds(base + 2 * BLKE, BLKE)], in_v, isem
                ).start()

        return carry

    lax.fori_loop(0, NPAIR, pair, 0)

    for b in range(2):
        base = elem0 + (STEPS - 2 + b) * BLKE
        pltpu.make_async_copy(
            bufs[b][1], out_hbm.at[pl.ds(base, BLKE)], bufs[b][3]).wait()


@jax.jit
def _permute(x, perm):
    mesh = plsc.VectorSubcoreMesh(core_axis_name="c", subcore_axis_name="s")
    run = pl.kernel(
        _permute_body,
        mesh=mesh,
        compiler_params=pltpu.CompilerParams(needs_layout_passes=False),
        out_type=jax.ShapeDtypeStruct((ROWS * COLS,), jnp.float32),
        scratch_types=[
            pltpu.VMEM((COLS,), jnp.int32),
            pltpu.VMEM((BLKE,), jnp.float32),
            pltpu.VMEM((BLKE,), jnp.float32),
            pltpu.VMEM((BLKE,), jnp.float32),
            pltpu.VMEM((BLKE,), jnp.float32),
            pltpu.SemaphoreType.DMA,
            pltpu.SemaphoreType.DMA,
            pltpu.SemaphoreType.DMA,
            pltpu.SemaphoreType.DMA,
        ],
    )
    return run(x.reshape(-1), perm).reshape(ROWS, COLS)


def kernel(x, perm, perm_inv):
    out = _permute(x, perm.astype(jnp.int32))
    return (out, 0)


# speedup vs baseline: 1.8224x; 1.5957x over previous
import jax
import jax.numpy as jnp
from jax import lax
from jax.experimental import pallas as pl
from jax.experimental.pallas import tpu as pltpu
from jax.experimental.pallas import tpu_sc as plsc

ROWS = 16384
COLS = 2048
LANES = 16
NUM_WORKERS = 32
ROWS_PER_W = ROWS // NUM_WORKERS
BLK = 8
STEPS = ROWS_PER_W // BLK
NPAIR = STEPS // 2
CHUNKS = COLS // LANES


def _permute_body(x_hbm, perm_hbm, out_hbm, perm_v,
                  in_a, in_b, out_a, out_b,
                  isem_a, isem_b, osem_a, osem_b):
    core = lax.axis_index("c")
    sub = lax.axis_index("s")
    wid = sub * 2 + core
    row0 = wid * ROWS_PER_W

    pltpu.sync_copy(perm_hbm, perm_v)

    bufs = ((in_a, out_a, isem_a, osem_a), (in_b, out_b, isem_b, osem_b))

    for b in range(2):
        pltpu.make_async_copy(
            x_hbm.at[pl.ds(row0 + b * BLK, BLK)], bufs[b][0], bufs[b][2]
        ).start()

    row_ids = [jnp.full((LANES,), r, dtype=jnp.int32) for r in range(BLK)]

    def pair(p, carry):
        for b in range(2):
            in_v, out_v, isem, osem = bufs[b]
            base = row0 + (p * 2 + b) * BLK
            pltpu.make_async_copy(
                x_hbm.at[pl.ds(base, BLK)], in_v, isem).wait()

            @pl.when(p >= 1)
            def _drain():
                pltpu.make_async_copy(
                    out_v, out_hbm.at[pl.ds(base - 2 * BLK, BLK)], osem
                ).wait()

            def chunk(c, inner):
                col = c * LANES
                idx = perm_v[pl.ds(col, LANES)]
                for r in range(BLK):
                    vals = plsc.load_gather(in_v, [row_ids[r], idx])
                    out_v[r, pl.ds(col, LANES)] = vals
                return inner

            lax.fori_loop(0, CHUNKS, chunk, 0, unroll=4)

            pltpu.make_async_copy(
                out_v, out_hbm.at[pl.ds(base, BLK)], osem).start()

            @pl.when(p + 1 < NPAIR)
            def _prefetch():
                pltpu.make_async_copy(
                    x_hbm.at[pl.ds(base + 2 * BLK, BLK)], in_v, isem
                ).start()

        return carry

    lax.fori_loop(0, NPAIR, pair, 0)

    for b in range(2):
        base = row0 + (STEPS - 2 + b) * BLK
        pltpu.make_async_copy(
            bufs[b][1], out_hbm.at[pl.ds(base, BLK)], bufs[b][3]).wait()


@jax.jit
def _permute(x, perm):
    mesh = plsc.VectorSubcoreMesh(core_axis_name="c", subcore_axis_name="s")
    run = pl.kernel(
        _permute_body,
        mesh=mesh,
        compiler_params=pltpu.CompilerParams(needs_layout_passes=False),
        out_type=jax.ShapeDtypeStruct((ROWS, COLS), jnp.float32),
        scratch_types=[
            pltpu.VMEM((COLS,), jnp.int32),
            pltpu.VMEM((BLK, COLS), jnp.float32),
            pltpu.VMEM((BLK, COLS), jnp.float32),
            pltpu.VMEM((BLK, COLS), jnp.float32),
            pltpu.VMEM((BLK, COLS), jnp.float32),
            pltpu.SemaphoreType.DMA,
            pltpu.SemaphoreType.DMA,
            pltpu.SemaphoreType.DMA,
            pltpu.SemaphoreType.DMA,
        ],
    )
    return run(x, perm)


def kernel(x, perm, perm_inv):
    out = _permute(x, perm.astype(jnp.int32))
    return (out, 0)


# speedup vs baseline: 5.4794x; 3.0067x over previous
import jax
import jax.numpy as jnp
from jax import lax
from jax.experimental import pallas as pl
from jax.experimental.pallas import tpu as pltpu
from jax.experimental.pallas import tpu_sc as plsc

ROWS = 16384
COLS = 2048
LANES = 16
NUM_WORKERS = 32
ROWS_PER_W = ROWS // NUM_WORKERS
BLK = 8
STEPS = ROWS_PER_W // BLK
NPAIR = STEPS // 2
CHUNKS = COLS // LANES


def _permute_body(x_hbm, perm_hbm, out_hbm, perm_v,
                  in_a, in_b, out_a, out_b,
                  isem_a, isem_b, osem_a, osem_b):
    core = lax.axis_index("c")
    sub = lax.axis_index("s")
    wid = sub * 2 + core
    row0 = wid * ROWS_PER_W

    pltpu.sync_copy(perm_hbm, perm_v)

    bufs = ((in_a, out_a, isem_a, osem_a), (in_b, out_b, isem_b, osem_b))

    for b in range(2):
        pltpu.make_async_copy(
            x_hbm.at[pl.ds(row0 + b * BLK, BLK)], bufs[b][0], bufs[b][2]
        ).start()

    row_ids = [jnp.full((LANES,), r, dtype=jnp.int32) for r in range(BLK)]

    def pair(p, carry):
        for b in range(2):
            in_v, out_v, isem, osem = bufs[b]
            base = row0 + (p * 2 + b) * BLK
            pltpu.make_async_copy(
                x_hbm.at[pl.ds(base, BLK)], in_v, isem).wait()

            @pl.when(p >= 1)
            def _drain():
                pltpu.make_async_copy(
                    out_v, out_hbm.at[pl.ds(base - 2 * BLK, BLK)], osem
                ).wait()

            @plsc.parallel_loop(0, CHUNKS, unroll=4)
            def chunk(c):
                col = c * LANES
                idx = perm_v[pl.ds(col, LANES)]
                for r in range(BLK):
                    vals = plsc.load_gather(in_v, [row_ids[r], idx])
                    out_v[r, pl.ds(col, LANES)] = vals

            pltpu.make_async_copy(
                out_v, out_hbm.at[pl.ds(base, BLK)], osem).start()

            @pl.when(p + 1 < NPAIR)
            def _prefetch():
                pltpu.make_async_copy(
                    x_hbm.at[pl.ds(base + 2 * BLK, BLK)], in_v, isem
                ).start()

        return carry

    lax.fori_loop(0, NPAIR, pair, 0)

    for b in range(2):
        base = row0 + (STEPS - 2 + b) * BLK
        pltpu.make_async_copy(
            bufs[b][1], out_hbm.at[pl.ds(base, BLK)], bufs[b][3]).wait()


@jax.jit
def _permute(x, perm):
    mesh = plsc.VectorSubcoreMesh(core_axis_name="c", subcore_axis_name="s")
    run = pl.kernel(
        _permute_body,
        mesh=mesh,
        compiler_params=pltpu.CompilerParams(needs_layout_passes=False),
        out_type=jax.ShapeDtypeStruct((ROWS, COLS), jnp.float32),
        scratch_types=[
            pltpu.VMEM((COLS,), jnp.int32),
            pltpu.VMEM((BLK, COLS), jnp.float32),
            pltpu.VMEM((BLK, COLS), jnp.float32),
            pltpu.VMEM((BLK, COLS), jnp.float32),
            pltpu.VMEM((BLK, COLS), jnp.float32),
            pltpu.SemaphoreType.DMA,
            pltpu.SemaphoreType.DMA,
            pltpu.SemaphoreType.DMA,
            pltpu.SemaphoreType.DMA,
        ],
    )
    return run(x, perm)


def kernel(x, perm, perm_inv):
    out = _permute(x, perm.astype(jnp.int32))
    return (out, 0)


# speedup vs baseline: 5.4882x; 1.0016x over previous
import jax
import jax.numpy as jnp
from jax import lax
from jax.experimental import pallas as pl
from jax.experimental.pallas import tpu as pltpu
from jax.experimental.pallas import tpu_sc as plsc

ROWS = 16384
COLS = 2048
LANES = 16
NUM_WORKERS = 32
ROWS_PER_W = ROWS // NUM_WORKERS
BLK = 8
STEPS = ROWS_PER_W // BLK
NPAIR = STEPS // 2
CHUNKS = COLS // LANES


def _permute_body(x_hbm, perm_hbm, out_hbm, perm_v,
                  in_a, in_b, out_a, out_b,
                  isem_a, isem_b, osem_a, osem_b):
    core = lax.axis_index("c")
    sub = lax.axis_index("s")
    wid = sub * 2 + core
    row0 = wid * ROWS_PER_W

    pltpu.sync_copy(perm_hbm, perm_v)

    bufs = ((in_a, out_a, isem_a, osem_a), (in_b, out_b, isem_b, osem_b))

    for b in range(2):
        pltpu.make_async_copy(
            x_hbm.at[pl.ds(row0 + b * BLK, BLK)], bufs[b][0], bufs[b][2]
        ).start()

    row_ids = [jnp.full((LANES,), r, dtype=jnp.int32) for r in range(BLK)]

    def pair(p, carry):
        for b in range(2):
            in_v, out_v, isem, osem = bufs[b]
            base = row0 + (p * 2 + b) * BLK
            pltpu.make_async_copy(
                x_hbm.at[pl.ds(base, BLK)], in_v, isem).wait()

            @pl.when(p >= 1)
            def _drain():
                pltpu.make_async_copy(
                    out_v, out_hbm.at[pl.ds(base - 2 * BLK, BLK)], osem
                ).wait()

            @plsc.parallel_loop(0, CHUNKS, unroll=8)
            def chunk(c):
                col = c * LANES
                idx = perm_v[pl.ds(col, LANES)]
                for r in range(BLK):
                    vals = plsc.load_gather(in_v, [row_ids[r], idx])
                    out_v[r, pl.ds(col, LANES)] = vals

            pltpu.make_async_copy(
                out_v, out_hbm.at[pl.ds(base, BLK)], osem).start()

            @pl.when(p + 1 < NPAIR)
            def _prefetch():
                pltpu.make_async_copy(
                    x_hbm.at[pl.ds(base + 2 * BLK, BLK)], in_v, isem
                ).start()

        return carry

    lax.fori_loop(0, NPAIR, pair, 0)

    for b in range(2):
        base = row0 + (STEPS - 2 + b) * BLK
        pltpu.make_async_copy(
            bufs[b][1], out_hbm.at[pl.ds(base, BLK)], bufs[b][3]).wait()


@jax.jit
def _permute(x, perm):
    mesh = plsc.VectorSubcoreMesh(core_axis_name="c", subcore_axis_name="s")
    run = pl.kernel(
        _permute_body,
        mesh=mesh,
        compiler_params=pltpu.CompilerParams(needs_layout_passes=False),
        out_type=jax.ShapeDtypeStruct((ROWS, COLS), jnp.float32),
        scratch_types=[
            pltpu.VMEM((COLS,), jnp.int32),
            pltpu.VMEM((BLK, COLS), jnp.float32),
            pltpu.VMEM((BLK, COLS), jnp.float32),
            pltpu.VMEM((BLK, COLS), jnp.float32),
            pltpu.VMEM((BLK, COLS), jnp.float32),
            pltpu.SemaphoreType.DMA,
            pltpu.SemaphoreType.DMA,
            pltpu.SemaphoreType.DMA,
            pltpu.SemaphoreType.DMA,
        ],
    )
    return run(x, perm)


def kernel(x, perm, perm_inv):
    out = _permute(x, perm.astype(jnp.int32))
    return (out, 0)
